# baseline (speedup 1.0000x reference)
import jax
import jax.numpy as jnp
from jax.experimental import pallas as pl
from jax.experimental.pallas import tpu as pltpu

_N = 16384
_D = 64
_K = 1024
_MATCH_T = 0.1
_SIM_T = 0.5

_NB = 8
_BN = _N // _NB
_RB = 128
_NR = _K // _RB


def _stats_body(y0_ref, y1_ref, d0_ref, d1_ref,
                cnt0_ref, cnt1_ref, csum0_ref, csum1_ref, cooc_ref):
    pi = pl.program_id(0)

    @pl.when(pi == 0)
    def _init():
        cnt0_ref[...] = jnp.zeros_like(cnt0_ref)
        cnt1_ref[...] = jnp.zeros_like(cnt1_ref)
        csum0_ref[...] = jnp.zeros_like(csum0_ref)
        csum1_ref[...] = jnp.zeros_like(csum1_ref)
        cooc_ref[...] = jnp.zeros_like(cooc_ref)

    iota = jax.lax.broadcasted_iota(jnp.int32, (_BN, _K), 1)
    oh0 = (y0_ref[0, 0, :][:, None] == iota).astype(jnp.bfloat16)
    oh1 = (y1_ref[0, 0, :][:, None] == iota).astype(jnp.bfloat16)

    dn = (((0,), (0,)), ((), ()))
    ones = jnp.ones((_BN, 1), dtype=jnp.bfloat16)
    cnt0_ref[...] += jax.lax.dot_general(
        ones, oh0, dn, preferred_element_type=jnp.float32)
    cnt1_ref[...] += jax.lax.dot_general(
        ones, oh1, dn, preferred_element_type=jnp.float32)
    for csum_ref, oh, d_ref in ((csum0_ref, oh0, d0_ref),
                                (csum1_ref, oh1, d1_ref)):
        d = d_ref[...]
        dhi = d.astype(jnp.bfloat16)
        dlo = (d - dhi.astype(jnp.float32)).astype(jnp.bfloat16)
        csum_ref[...] += (
            jax.lax.dot_general(oh, dhi, dn,
                                preferred_element_type=jnp.float32)
            + jax.lax.dot_general(oh, dlo, dn,
                                  preferred_element_type=jnp.float32))
    cooc_ref[...] += jax.lax.dot_general(
        oh0, oh1, dn, preferred_element_type=jnp.float32)


def _radius_body(y0_ref, y1_ref, d0_ref, d1_ref, cnt0_ref, cnt1_ref,
                 csum0_ref, csum1_ref, rsum0_ref, rsum1_ref):
    pi = pl.program_id(0)

    @pl.when(pi == 0)
    def _init():
        rsum0_ref[...] = jnp.zeros_like(rsum0_ref)
        rsum1_ref[...] = jnp.zeros_like(rsum1_ref)

    iota = jax.lax.broadcasted_iota(jnp.int32, (_BN, _K), 1)
    dn = (((0,), (0,)), ((), ()))

    def one_view(y_ref, d_ref, cnt_ref, csum_ref, rsum_ref):
        rc = 1.0 / jnp.clip(cnt_ref[...], 1e-6, None)
        centers = csum_ref[...] * rc.T
        oh = (y_ref[0, 0, :][:, None] == iota).astype(jnp.bfloat16)
        chi = centers.astype(jnp.bfloat16)
        clo = (centers - chi.astype(jnp.float32)).astype(jnp.bfloat16)
        gathered = (jnp.dot(oh, chi, preferred_element_type=jnp.float32)
                    + jnp.dot(oh, clo, preferred_element_type=jnp.float32))
        diff = d_ref[...] - gathered
        d = jnp.sqrt(jnp.sum(diff * diff, axis=1, keepdims=True))
        dhi = d.astype(jnp.bfloat16)
        dlo = (d - dhi.astype(jnp.float32)).astype(jnp.bfloat16)
        rsum_ref[...] += (
            jax.lax.dot_general(dhi, oh, dn,
                                preferred_element_type=jnp.float32)
            + jax.lax.dot_general(dlo, oh, dn,
                                  preferred_element_type=jnp.float32)
        ).reshape(1, _K)

    one_view(y0_ref, d0_ref, cnt0_ref, csum0_ref, rsum0_ref)
    one_view(y1_ref, d1_ref, cnt1_ref, csum1_ref, rsum1_ref)


def _loss_body(cnt0_ref, cnt1_ref, csum0_ref, csum1_ref,
               rsum0_ref, rsum1_ref, cooc_ref, out_ref, acc_ref):
    pi = pl.program_id(0)

    @pl.when(pi == 0)
    def _init():
        acc_ref[0] = 0.0
        acc_ref[1] = 0.0

    cnt0 = cnt0_ref[...]
    cnt1 = cnt1_ref[...]
    rc0 = 1.0 / jnp.clip(cnt0, 1e-6, None)
    rc1 = 1.0 / jnp.clip(cnt1, 1e-6, None)
    centers0 = csum0_ref[...] * rc0.T
    centers1 = csum1_ref[...] * rc1.T
    rs0 = rsum0_ref[...] * rc0
    rs1 = rsum1_ref[...] * rc1

    x = jnp.concatenate([centers0, centers1], axis=0)
    nx = jnp.sqrt(jnp.sum(x * x, axis=1, keepdims=True))

    a0 = pi * _RB

    def half(centers, rs, csum_ref, rsum_ref, cnt_blk_ref, half_idx):
        cntb = cnt_blk_ref[:, pl.ds(a0, _RB)]
        rcb = 1.0 / jnp.clip(cntb, 1e-6, None)
        cb = csum_ref[pl.ds(a0, _RB), :] * rcb.T
        rsb = rsum_ref[:, pl.ds(a0, _RB)] * rcb

        aa = jnp.sum(cb * cb, axis=1, keepdims=True)
        bb = jnp.sum(centers * centers, axis=1, keepdims=True).T
        sq = aa + bb - 2.0 * jnp.dot(cb, centers.T,
                                     preferred_element_type=jnp.float32)
        dist = jnp.sqrt(jnp.clip(sq, 0.0, None))
        extra = rsb.T + rs
        geo = dist <= extra
        cnorm = jnp.clip(jnp.sqrt(bb), 1e-8, None)
        cnb = cb / jnp.clip(jnp.sqrt(aa), 1e-8, None)
        sim = jnp.dot(cnb, (centers / cnorm.T).T,
                      preferred_element_type=jnp.float32)
        mask = jnp.logical_and(geo, sim > _SIM_T).astype(jnp.float32)

        ca = cntb.T
        if half_idx == 0:
            coocb = cooc_ref[pl.ds(a0, _RB), :]
            m = jnp.minimum(ca, cnt1)
        else:
            coocb = cooc_ref[:, pl.ds(a0, _RB)].T
            m = jnp.minimum(ca, cnt0)
        m = jnp.where(m == 0.0, 1.0, m)
        inter = (coocb / m > _MATCH_T).astype(jnp.float32)

        if half_idx == 0:
            posb = jnp.concatenate([mask, inter], axis=1)
        else:
            posb = jnp.concatenate([inter, mask], axis=1)

        grow = a0 + half_idx * _K + jax.lax.broadcasted_iota(
            jnp.int32, (_RB, 2 * _K), 0)
        gcol = jax.lax.broadcasted_iota(jnp.int32, (_RB, 2 * _K), 1)
        posb = jnp.where(gcol == grow, 0.0, posb)

        nxb = jnp.sqrt(aa)
        num = jnp.dot(cb, x.T, preferred_element_type=jnp.float32)
        logits = num / (nxb * nx.T + 1e-12)
        mx = jnp.max(logits, axis=1, keepdims=True)
        lse = mx + jnp.log(jnp.sum(jnp.exp(logits - mx), axis=1,
                                   keepdims=True))
        log_prob = logits - lse

        pc = jnp.sum(posb, axis=1)
        valid = pc > 0.0
        rl = -jnp.sum(log_prob * posb, axis=1) / jnp.clip(pc, 1.0, None)
        contrib = jnp.sum(jnp.where(valid, rl, 0.0))
        nv = jnp.sum(valid.astype(jnp.float32))
        return contrib, nv

    c0, n0 = half(centers0, rs0, csum0_ref, rsum0_ref, cnt0_ref, 0)
    c1, n1 = half(centers1, rs1, csum1_ref, rsum1_ref, cnt1_ref, 1)
    acc_ref[0] += c0 + c1
    acc_ref[1] += n0 + n1

    @pl.when(pi == _NR - 1)
    def _fin():
        out_ref[0, 0] = acc_ref[0] / jnp.maximum(acc_ref[1], 1.0)


def kernel(data0, data1, y_parts0, y_parts1):
    y0 = y_parts0.reshape(_NB, 1, _BN)
    y1 = y_parts1.reshape(_NB, 1, _BN)

    yspec = pl.BlockSpec((1, 1, _BN), lambda i: (i, 0, 0))
    dspec = pl.BlockSpec((_BN, _D), lambda i: (i, 0))
    full = lambda shp: pl.BlockSpec(shp, lambda i: tuple(0 for _ in shp))

    cnt0, cnt1, csum0, csum1, cooc = pl.pallas_call(
        _stats_body,
        grid=(_NB,),
        in_specs=[yspec, yspec, dspec, dspec],
        out_specs=[full((1, _K)), full((1, _K)),
                   full((_K, _D)), full((_K, _D)), full((_K, _K))],
        out_shape=[
            jax.ShapeDtypeStruct((1, _K), jnp.float32),
            jax.ShapeDtypeStruct((1, _K), jnp.float32),
            jax.ShapeDtypeStruct((_K, _D), jnp.float32),
            jax.ShapeDtypeStruct((_K, _D), jnp.float32),
            jax.ShapeDtypeStruct((_K, _K), jnp.float32),
        ],
        compiler_params=pltpu.CompilerParams(
            dimension_semantics=("arbitrary",)),
    )(y0, y1, data0, data1)

    rsum0, rsum1 = pl.pallas_call(
        _radius_body,
        grid=(_NB,),
        in_specs=[yspec, yspec, dspec, dspec,
                  full((1, _K)), full((1, _K)),
                  full((_K, _D)), full((_K, _D))],
        out_specs=[full((1, _K)), full((1, _K))],
        out_shape=[
            jax.ShapeDtypeStruct((1, _K), jnp.float32),
            jax.ShapeDtypeStruct((1, _K), jnp.float32),
        ],
        compiler_params=pltpu.CompilerParams(
            dimension_semantics=("arbitrary",)),
    )(y0, y1, data0, data1, cnt0, cnt1, csum0, csum1)

    loss = pl.pallas_call(
        _loss_body,
        grid=(_NR,),
        in_specs=[full((1, _K)), full((1, _K)),
                  full((_K, _D)), full((_K, _D)),
                  full((1, _K)), full((1, _K)), full((_K, _K))],
        out_specs=pl.BlockSpec(memory_space=pltpu.SMEM),
        out_shape=jax.ShapeDtypeStruct((1, 1), jnp.float32),
        scratch_shapes=[pltpu.SMEM((2,), jnp.float32)],
        compiler_params=pltpu.CompilerParams(
            dimension_semantics=("arbitrary",)),
    )(cnt0, cnt1, csum0, csum1, rsum0, rsum1, cooc)

    return loss[0, 0]


# speedup vs baseline: 1.4069x; 1.4069x over previous
import jax
import jax.numpy as jnp
from jax.experimental import pallas as pl
from jax.experimental.pallas import tpu as pltpu

_N = 16384
_D = 64
_K = 1024
_MATCH_T = 0.1
_SIM_T = 0.5

_NB = 8
_BN = _N // _NB
_RB = 128
_NR = _K // _RB


def _stats_body(y0_ref, y1_ref, d0_ref, d1_ref,
                cnt0_ref, cnt1_ref, csum0_ref, csum1_ref, cooc_ref):
    pi = pl.program_id(0)

    @pl.when(pi == 0)
    def _init():
        cnt0_ref[...] = jnp.zeros_like(cnt0_ref)
        cnt1_ref[...] = jnp.zeros_like(cnt1_ref)
        csum0_ref[...] = jnp.zeros_like(csum0_ref)
        csum1_ref[...] = jnp.zeros_like(csum1_ref)
        cooc_ref[...] = jnp.zeros_like(cooc_ref)

    iota = jax.lax.broadcasted_iota(jnp.int32, (_BN, _K), 1)
    oh0 = (y0_ref[0, 0, :][:, None] == iota).astype(jnp.float32)
    oh1 = (y1_ref[0, 0, :][:, None] == iota).astype(jnp.float32)

    cnt0_ref[...] += jnp.sum(oh0, axis=0, keepdims=True)
    cnt1_ref[...] += jnp.sum(oh1, axis=0, keepdims=True)
    dn = (((0,), (0,)), ((), ()))
    csum0_ref[...] += jax.lax.dot_general(
        oh0, d0_ref[...], dn, preferred_element_type=jnp.float32)
    csum1_ref[...] += jax.lax.dot_general(
        oh1, d1_ref[...], dn, preferred_element_type=jnp.float32)
    cooc_ref[...] += jax.lax.dot_general(
        oh0, oh1, dn, preferred_element_type=jnp.float32)


def _radius_body(y0_ref, y1_ref, d0_ref, d1_ref, cnt0_ref, cnt1_ref,
                 csum0_ref, csum1_ref, rsum0_ref, rsum1_ref):
    pi = pl.program_id(0)

    @pl.when(pi == 0)
    def _init():
        rsum0_ref[...] = jnp.zeros_like(rsum0_ref)
        rsum1_ref[...] = jnp.zeros_like(rsum1_ref)

    iota = jax.lax.broadcasted_iota(jnp.int32, (_BN, _K), 1)
    dn = (((0,), (0,)), ((), ()))

    def one_view(y_ref, d_ref, cnt_ref, csum_ref, rsum_ref):
        rc = 1.0 / jnp.clip(cnt_ref[...], 1e-6, None)
        centers = csum_ref[...] * rc.T
        oh = (y_ref[0, 0, :][:, None] == iota).astype(jnp.float32)
        gathered = jnp.dot(oh, centers, preferred_element_type=jnp.float32)
        diff = d_ref[...] - gathered
        d = jnp.sqrt(jnp.sum(diff * diff, axis=1, keepdims=True))
        rsum_ref[...] += jax.lax.dot_general(
            d, oh, dn, preferred_element_type=jnp.float32).reshape(1, _K)

    one_view(y0_ref, d0_ref, cnt0_ref, csum0_ref, rsum0_ref)
    one_view(y1_ref, d1_ref, cnt1_ref, csum1_ref, rsum1_ref)


def _loss_body(cnt0_ref, cnt1_ref, csum0_ref, csum1_ref,
               rsum0_ref, rsum1_ref, cooc_ref, out_ref, acc_ref):
    pi = pl.program_id(0)

    @pl.when(pi == 0)
    def _init():
        acc_ref[0] = 0.0
        acc_ref[1] = 0.0

    cnt0 = cnt0_ref[...]
    cnt1 = cnt1_ref[...]
    rc0 = 1.0 / jnp.clip(cnt0, 1e-6, None)
    rc1 = 1.0 / jnp.clip(cnt1, 1e-6, None)
    centers0 = csum0_ref[...] * rc0.T
    centers1 = csum1_ref[...] * rc1.T
    rs0 = rsum0_ref[...] * rc0
    rs1 = rsum1_ref[...] * rc1

    x = jnp.concatenate([centers0, centers1], axis=0)
    nx = jnp.sqrt(jnp.sum(x * x, axis=1, keepdims=True))

    a0 = pi * _RB

    def half(centers, rs, csum_ref, rsum_ref, cnt_blk_ref, half_idx):
        cntb = cnt_blk_ref[:, pl.ds(a0, _RB)]
        rcb = 1.0 / jnp.clip(cntb, 1e-6, None)
        cb = csum_ref[pl.ds(a0, _RB), :] * rcb.T
        rsb = rsum_ref[:, pl.ds(a0, _RB)] * rcb

        aa = jnp.sum(cb * cb, axis=1, keepdims=True)
        bb = jnp.sum(centers * centers, axis=1, keepdims=True).T
        sq = aa + bb - 2.0 * jnp.dot(cb, centers.T,
                                     preferred_element_type=jnp.float32)
        dist = jnp.sqrt(jnp.clip(sq, 0.0, None))
        extra = rsb.T + rs
        geo = dist <= extra
        cnorm = jnp.clip(jnp.sqrt(bb), 1e-8, None)
        cnb = cb / jnp.clip(jnp.sqrt(aa), 1e-8, None)
        sim = jnp.dot(cnb, (centers / cnorm.T).T,
                      preferred_element_type=jnp.float32)
        mask = jnp.logical_and(geo, sim > _SIM_T).astype(jnp.float32)

        ca = cntb.T
        if half_idx == 0:
            coocb = cooc_ref[pl.ds(a0, _RB), :]
            m = jnp.minimum(ca, cnt1)
        else:
            coocb = cooc_ref[:, pl.ds(a0, _RB)].T
            m = jnp.minimum(ca, cnt0)
        m = jnp.where(m == 0.0, 1.0, m)
        inter = (coocb / m > _MATCH_T).astype(jnp.float32)

        if half_idx == 0:
            posb = jnp.concatenate([mask, inter], axis=1)
        else:
            posb = jnp.concatenate([inter, mask], axis=1)

        grow = a0 + half_idx * _K + jax.lax.broadcasted_iota(
            jnp.int32, (_RB, 2 * _K), 0)
        gcol = jax.lax.broadcasted_iota(jnp.int32, (_RB, 2 * _K), 1)
        posb = jnp.where(gcol == grow, 0.0, posb)

        nxb = jnp.sqrt(aa)
        num = jnp.dot(cb, x.T, preferred_element_type=jnp.float32)
        logits = num / (nxb * nx.T + 1e-12)
        mx = jnp.max(logits, axis=1, keepdims=True)
        lse = mx + jnp.log(jnp.sum(jnp.exp(logits - mx), axis=1,
                                   keepdims=True))
        log_prob = logits - lse

        pc = jnp.sum(posb, axis=1)
        valid = pc > 0.0
        rl = -jnp.sum(log_prob * posb, axis=1) / jnp.clip(pc, 1.0, None)
        contrib = jnp.sum(jnp.where(valid, rl, 0.0))
        nv = jnp.sum(valid.astype(jnp.float32))
        return contrib, nv

    c0, n0 = half(centers0, rs0, csum0_ref, rsum0_ref, cnt0_ref, 0)
    c1, n1 = half(centers1, rs1, csum1_ref, rsum1_ref, cnt1_ref, 1)
    acc_ref[0] += c0 + c1
    acc_ref[1] += n0 + n1

    @pl.when(pi == _NR - 1)
    def _fin():
        out_ref[0, 0] = acc_ref[0] / jnp.maximum(acc_ref[1], 1.0)


def kernel(data0, data1, y_parts0, y_parts1):
    y0 = y_parts0.reshape(_NB, 1, _BN)
    y1 = y_parts1.reshape(_NB, 1, _BN)

    yspec = pl.BlockSpec((1, 1, _BN), lambda i: (i, 0, 0))
    dspec = pl.BlockSpec((_BN, _D), lambda i: (i, 0))
    full = lambda shp: pl.BlockSpec(shp, lambda i: tuple(0 for _ in shp))

    cnt0, cnt1, csum0, csum1, cooc = pl.pallas_call(
        _stats_body,
        grid=(_NB,),
        in_specs=[yspec, yspec, dspec, dspec],
        out_specs=[full((1, _K)), full((1, _K)),
                   full((_K, _D)), full((_K, _D)), full((_K, _K))],
        out_shape=[
            jax.ShapeDtypeStruct((1, _K), jnp.float32),
            jax.ShapeDtypeStruct((1, _K), jnp.float32),
            jax.ShapeDtypeStruct((_K, _D), jnp.float32),
            jax.ShapeDtypeStruct((_K, _D), jnp.float32),
            jax.ShapeDtypeStruct((_K, _K), jnp.float32),
        ],
        compiler_params=pltpu.CompilerParams(
            dimension_semantics=("arbitrary",)),
    )(y0, y1, data0, data1)

    rsum0, rsum1 = pl.pallas_call(
        _radius_body,
        grid=(_NB,),
        in_specs=[yspec, yspec, dspec, dspec,
                  full((1, _K)), full((1, _K)),
                  full((_K, _D)), full((_K, _D))],
        out_specs=[full((1, _K)), full((1, _K))],
        out_shape=[
            jax.ShapeDtypeStruct((1, _K), jnp.float32),
            jax.ShapeDtypeStruct((1, _K), jnp.float32),
        ],
        compiler_params=pltpu.CompilerParams(
            dimension_semantics=("arbitrary",)),
    )(y0, y1, data0, data1, cnt0, cnt1, csum0, csum1)

    loss = pl.pallas_call(
        _loss_body,
        grid=(_NR,),
        in_specs=[full((1, _K)), full((1, _K)),
                  full((_K, _D)), full((_K, _D)),
                  full((1, _K)), full((1, _K)), full((_K, _K))],
        out_specs=pl.BlockSpec(memory_space=pltpu.SMEM),
        out_shape=jax.ShapeDtypeStruct((1, 1), jnp.float32),
        scratch_shapes=[pltpu.SMEM((2,), jnp.float32)],
        compiler_params=pltpu.CompilerParams(
            dimension_semantics=("arbitrary",)),
    )(cnt0, cnt1, csum0, csum1, rsum0, rsum1, cooc)

    return loss[0, 0]


# speedup vs baseline: 1.4422x; 1.0251x over previous
import jax
import jax.numpy as jnp
from jax.experimental import pallas as pl
from jax.experimental.pallas import tpu as pltpu

_N = 16384
_D = 64
_K = 1024
_MATCH_T = 0.1
_SIM_T = 0.5

_NB = 8
_BN = _N // _NB
_RB = 128
_NR = _K // _RB


def _body(y0_ref, y1_ref, d0_ref, d1_ref,
          cnt0_ref, cnt1_ref, csum0_ref, csum1_ref, cooc_ref,
          rsum0_ref, rsum1_ref, out_ref, acc_ref):
    i = pl.program_id(0)
    dn = (((0,), (0,)), ((), ()))

    @pl.when(i == 0)
    def _init():
        cnt0_ref[...] = jnp.zeros_like(cnt0_ref)
        cnt1_ref[...] = jnp.zeros_like(cnt1_ref)
        csum0_ref[...] = jnp.zeros_like(csum0_ref)
        csum1_ref[...] = jnp.zeros_like(csum1_ref)
        cooc_ref[...] = jnp.zeros_like(cooc_ref)
        rsum0_ref[...] = jnp.zeros_like(rsum0_ref)
        rsum1_ref[...] = jnp.zeros_like(rsum1_ref)
        acc_ref[0] = 0.0
        acc_ref[1] = 0.0

    @pl.when(i < _NB)
    def _stats():
        iota = jax.lax.broadcasted_iota(jnp.int32, (_BN, _K), 1)
        oh0 = (y0_ref[0, 0, :][:, None] == iota).astype(jnp.float32)
        oh1 = (y1_ref[0, 0, :][:, None] == iota).astype(jnp.float32)
        cnt0_ref[...] += jnp.sum(oh0, axis=0, keepdims=True)
        cnt1_ref[...] += jnp.sum(oh1, axis=0, keepdims=True)
        csum0_ref[...] += jax.lax.dot_general(
            oh0, d0_ref[...], dn, preferred_element_type=jnp.float32)
        csum1_ref[...] += jax.lax.dot_general(
            oh1, d1_ref[...], dn, preferred_element_type=jnp.float32)
        cooc_ref[...] += jax.lax.dot_general(
            oh0, oh1, dn, preferred_element_type=jnp.float32)

    @pl.when(jnp.logical_and(i >= _NB, i < 2 * _NB))
    def _radius():
        iota = jax.lax.broadcasted_iota(jnp.int32, (_BN, _K), 1)

        def one_view(y_ref, d_ref, cnt_ref, csum_ref, rsum_ref):
            rc = 1.0 / jnp.clip(cnt_ref[...], 1e-6, None)
            centers = csum_ref[...] * rc.T
            oh = (y_ref[0, 0, :][:, None] == iota).astype(jnp.float32)
            gathered = jnp.dot(oh, centers,
                               preferred_element_type=jnp.float32)
            diff = d_ref[...] - gathered
            d = jnp.sqrt(jnp.sum(diff * diff, axis=1, keepdims=True))
            rsum_ref[...] += jax.lax.dot_general(
                d, oh, dn, preferred_element_type=jnp.float32
            ).reshape(1, _K)

        one_view(y0_ref, d0_ref, cnt0_ref, csum0_ref, rsum0_ref)
        one_view(y1_ref, d1_ref, cnt1_ref, csum1_ref, rsum1_ref)

    @pl.when(i >= 2 * _NB)
    def _loss():
        pi = i - 2 * _NB
        cnt0 = cnt0_ref[...]
        cnt1 = cnt1_ref[...]
        rc0 = 1.0 / jnp.clip(cnt0, 1e-6, None)
        rc1 = 1.0 / jnp.clip(cnt1, 1e-6, None)
        centers0 = csum0_ref[...] * rc0.T
        centers1 = csum1_ref[...] * rc1.T
        rs0 = rsum0_ref[...] * rc0
        rs1 = rsum1_ref[...] * rc1

        x = jnp.concatenate([centers0, centers1], axis=0)
        nx = jnp.sqrt(jnp.sum(x * x, axis=1, keepdims=True))

        a0 = pi * _RB

        def half(centers, rs, csum_ref, rsum_ref, cnt_blk_ref, half_idx):
            cntb = cnt_blk_ref[:, pl.ds(a0, _RB)]
            rcb = 1.0 / jnp.clip(cntb, 1e-6, None)
            cb = csum_ref[pl.ds(a0, _RB), :] * rcb.T
            rsb = rsum_ref[:, pl.ds(a0, _RB)] * rcb

            aa = jnp.sum(cb * cb, axis=1, keepdims=True)
            bb = jnp.sum(centers * centers, axis=1,
                         keepdims=True).T
            sq = aa + bb - 2.0 * jnp.dot(
                cb, centers.T, preferred_element_type=jnp.float32)
            dist = jnp.sqrt(jnp.clip(sq, 0.0, None))
            extra = rsb.T + rs
            geo = dist <= extra
            cnorm = jnp.clip(jnp.sqrt(bb), 1e-8, None)
            cnb = cb / jnp.clip(jnp.sqrt(aa), 1e-8, None)
            sim = jnp.dot(cnb, (centers / cnorm.T).T,
                          preferred_element_type=jnp.float32)
            mask = jnp.logical_and(geo, sim > _SIM_T).astype(jnp.float32)

            ca = cntb.T
            if half_idx == 0:
                coocb = cooc_ref[pl.ds(a0, _RB), :]
                m = jnp.minimum(ca, cnt1)
            else:
                coocb = cooc_ref[:, pl.ds(a0, _RB)].T
                m = jnp.minimum(ca, cnt0)
            m = jnp.where(m == 0.0, 1.0, m)
            inter = (coocb / m > _MATCH_T).astype(jnp.float32)

            if half_idx == 0:
                posb = jnp.concatenate([mask, inter], axis=1)
            else:
                posb = jnp.concatenate([inter, mask], axis=1)

            grow = a0 + half_idx * _K + jax.lax.broadcasted_iota(
                jnp.int32, (_RB, 2 * _K), 0)
            gcol = jax.lax.broadcasted_iota(jnp.int32, (_RB, 2 * _K), 1)
            posb = jnp.where(gcol == grow, 0.0, posb)

            nxb = jnp.sqrt(aa)
            num = jnp.dot(cb, x.T, preferred_element_type=jnp.float32)
            logits = num / (nxb * nx.T + 1e-12)
            mx = jnp.max(logits, axis=1, keepdims=True)
            lse = mx + jnp.log(jnp.sum(jnp.exp(logits - mx), axis=1,
                                       keepdims=True))
            log_prob = logits - lse

            pc = jnp.sum(posb, axis=1)
            valid = pc > 0.0
            rl = -jnp.sum(log_prob * posb, axis=1) / jnp.clip(pc, 1.0,
                                                              None)
            contrib = jnp.sum(jnp.where(valid, rl, 0.0))
            nv = jnp.sum(valid.astype(jnp.float32))
            return contrib, nv

        c0, n0 = half(centers0, rs0, csum0_ref, rsum0_ref, cnt0_ref, 0)
        c1, n1 = half(centers1, rs1, csum1_ref, rsum1_ref, cnt1_ref, 1)
        acc_ref[0] += c0 + c1
        acc_ref[1] += n0 + n1

    @pl.when(i == 2 * _NB + _NR - 1)
    def _fin():
        out_ref[0, 0] = acc_ref[0] / jnp.maximum(acc_ref[1], 1.0)


def kernel(data0, data1, y_parts0, y_parts1):
    y0 = y_parts0.reshape(_NB, 1, _BN)
    y1 = y_parts1.reshape(_NB, 1, _BN)

    yspec = pl.BlockSpec((1, 1, _BN), lambda i: (i % _NB, 0, 0))
    dspec = pl.BlockSpec((_BN, _D), lambda i: (i % _NB, 0))
    full = lambda shp: pl.BlockSpec(shp, lambda i: tuple(0 for _ in shp))

    outs = pl.pallas_call(
        _body,
        grid=(2 * _NB + _NR,),
        in_specs=[yspec, yspec, dspec, dspec],
        out_specs=[full((1, _K)), full((1, _K)),
                   full((_K, _D)), full((_K, _D)), full((_K, _K)),
                   full((1, _K)), full((1, _K)),
                   pl.BlockSpec(memory_space=pltpu.SMEM)],
        out_shape=[
            jax.ShapeDtypeStruct((1, _K), jnp.float32),
            jax.ShapeDtypeStruct((1, _K), jnp.float32),
            jax.ShapeDtypeStruct((_K, _D), jnp.float32),
            jax.ShapeDtypeStruct((_K, _D), jnp.float32),
            jax.ShapeDtypeStruct((_K, _K), jnp.float32),
            jax.ShapeDtypeStruct((1, _K), jnp.float32),
            jax.ShapeDtypeStruct((1, _K), jnp.float32),
            jax.ShapeDtypeStruct((1, 1), jnp.float32),
        ],
        scratch_shapes=[pltpu.SMEM((2,), jnp.float32)],
        compiler_params=pltpu.CompilerParams(
            dimension_semantics=("arbitrary",)),
    )(y0, y1, data0, data1)

    return outs[-1][0, 0]


# speedup vs baseline: 1.4580x; 1.0110x over previous
import jax
import jax.numpy as jnp
from jax.experimental import pallas as pl
from jax.experimental.pallas import tpu as pltpu

_N = 16384
_D = 64
_K = 1024
_MATCH_T = 0.1
_SIM_T = 0.5

_NB = 8
_BN = _N // _NB
_RB = 128
_NR = _K // _RB


def _body(y0_ref, y1_ref, d0_ref, d1_ref, out_ref,
          cnt0_ref, cnt1_ref, csum0_ref, csum1_ref, cooc_ref,
          rsum0_ref, rsum1_ref, acc_ref):
    i = pl.program_id(0)
    dn = (((0,), (0,)), ((), ()))

    @pl.when(i == 0)
    def _init():
        cnt0_ref[...] = jnp.zeros_like(cnt0_ref)
        cnt1_ref[...] = jnp.zeros_like(cnt1_ref)
        csum0_ref[...] = jnp.zeros_like(csum0_ref)
        csum1_ref[...] = jnp.zeros_like(csum1_ref)
        cooc_ref[...] = jnp.zeros_like(cooc_ref)
        rsum0_ref[...] = jnp.zeros_like(rsum0_ref)
        rsum1_ref[...] = jnp.zeros_like(rsum1_ref)
        acc_ref[0] = 0.0
        acc_ref[1] = 0.0

    @pl.when(i < _NB)
    def _stats():
        iota = jax.lax.broadcasted_iota(jnp.int32, (_BN, _K), 1)
        oh0 = (y0_ref[0, 0, :][:, None] == iota).astype(jnp.float32)
        oh1 = (y1_ref[0, 0, :][:, None] == iota).astype(jnp.float32)
        cnt0_ref[...] += jnp.sum(oh0, axis=0, keepdims=True)
        cnt1_ref[...] += jnp.sum(oh1, axis=0, keepdims=True)
        csum0_ref[...] += jax.lax.dot_general(
            oh0, d0_ref[...], dn, preferred_element_type=jnp.float32)
        csum1_ref[...] += jax.lax.dot_general(
            oh1, d1_ref[...], dn, preferred_element_type=jnp.float32)
        cooc_ref[...] += jax.lax.dot_general(
            oh0, oh1, dn, preferred_element_type=jnp.float32)

    @pl.when(jnp.logical_and(i >= _NB, i < 2 * _NB))
    def _radius():
        iota = jax.lax.broadcasted_iota(jnp.int32, (_BN, _K), 1)

        def one_view(y_ref, d_ref, cnt_ref, csum_ref, rsum_ref):
            rc = 1.0 / jnp.clip(cnt_ref[...], 1e-6, None)
            centers = csum_ref[...] * rc.T
            oh = (y_ref[0, 0, :][:, None] == iota).astype(jnp.float32)
            gathered = jnp.dot(oh, centers,
                               preferred_element_type=jnp.float32)
            diff = d_ref[...] - gathered
            d = jnp.sqrt(jnp.sum(diff * diff, axis=1, keepdims=True))
            rsum_ref[...] += jax.lax.dot_general(
                d, oh, dn, preferred_element_type=jnp.float32
            ).reshape(1, _K)

        one_view(y0_ref, d0_ref, cnt0_ref, csum0_ref, rsum0_ref)
        one_view(y1_ref, d1_ref, cnt1_ref, csum1_ref, rsum1_ref)

    @pl.when(i >= 2 * _NB)
    def _loss():
        pi = i - 2 * _NB
        cnt0 = cnt0_ref[...]
        cnt1 = cnt1_ref[...]
        rc0 = 1.0 / jnp.clip(cnt0, 1e-6, None)
        rc1 = 1.0 / jnp.clip(cnt1, 1e-6, None)
        centers0 = csum0_ref[...] * rc0.T
        centers1 = csum1_ref[...] * rc1.T
        rs0 = rsum0_ref[...] * rc0
        rs1 = rsum1_ref[...] * rc1

        x = jnp.concatenate([centers0, centers1], axis=0)
        nx = jnp.sqrt(jnp.sum(x * x, axis=1, keepdims=True))

        a0 = pi * _RB

        def half(centers, rs, csum_ref, rsum_ref, cnt_blk_ref, half_idx):
            cntb = cnt_blk_ref[:, pl.ds(a0, _RB)]
            rcb = 1.0 / jnp.clip(cntb, 1e-6, None)
            cb = csum_ref[pl.ds(a0, _RB), :] * rcb.T
            rsb = rsum_ref[:, pl.ds(a0, _RB)] * rcb

            aa = jnp.sum(cb * cb, axis=1, keepdims=True)
            bb = jnp.sum(centers * centers, axis=1,
                         keepdims=True).T
            sq = aa + bb - 2.0 * jnp.dot(
                cb, centers.T, preferred_element_type=jnp.float32)
            dist = jnp.sqrt(jnp.clip(sq, 0.0, None))
            extra = rsb.T + rs
            geo = dist <= extra
            cnorm = jnp.clip(jnp.sqrt(bb), 1e-8, None)
            cnb = cb / jnp.clip(jnp.sqrt(aa), 1e-8, None)
            sim = jnp.dot(cnb, (centers / cnorm.T).T,
                          preferred_element_type=jnp.float32)
            mask = jnp.logical_and(geo, sim > _SIM_T).astype(jnp.float32)

            ca = cntb.T
            if half_idx == 0:
                coocb = cooc_ref[pl.ds(a0, _RB), :]
                m = jnp.minimum(ca, cnt1)
            else:
                coocb = cooc_ref[:, pl.ds(a0, _RB)].T
                m = jnp.minimum(ca, cnt0)
            m = jnp.where(m == 0.0, 1.0, m)
            inter = (coocb / m > _MATCH_T).astype(jnp.float32)

            if half_idx == 0:
                posb = jnp.concatenate([mask, inter], axis=1)
            else:
                posb = jnp.concatenate([inter, mask], axis=1)

            grow = a0 + half_idx * _K + jax.lax.broadcasted_iota(
                jnp.int32, (_RB, 2 * _K), 0)
            gcol = jax.lax.broadcasted_iota(jnp.int32, (_RB, 2 * _K), 1)
            posb = jnp.where(gcol == grow, 0.0, posb)

            nxb = jnp.sqrt(aa)
            num = jnp.dot(cb, x.T, preferred_element_type=jnp.float32)
            logits = num / (nxb * nx.T + 1e-12)
            mx = jnp.max(logits, axis=1, keepdims=True)
            lse = mx + jnp.log(jnp.sum(jnp.exp(logits - mx), axis=1,
                                       keepdims=True))
            log_prob = logits - lse

            pc = jnp.sum(posb, axis=1)
            valid = pc > 0.0
            rl = -jnp.sum(log_prob * posb, axis=1) / jnp.clip(pc, 1.0,
                                                              None)
            contrib = jnp.sum(jnp.where(valid, rl, 0.0))
            nv = jnp.sum(valid.astype(jnp.float32))
            return contrib, nv

        c0, n0 = half(centers0, rs0, csum0_ref, rsum0_ref, cnt0_ref, 0)
        c1, n1 = half(centers1, rs1, csum1_ref, rsum1_ref, cnt1_ref, 1)
        acc_ref[0] += c0 + c1
        acc_ref[1] += n0 + n1

    @pl.when(i == 2 * _NB + _NR - 1)
    def _fin():
        out_ref[0, 0] = acc_ref[0] / jnp.maximum(acc_ref[1], 1.0)


def kernel(data0, data1, y_parts0, y_parts1):
    y0 = y_parts0.reshape(_NB, 1, _BN)
    y1 = y_parts1.reshape(_NB, 1, _BN)

    yspec = pl.BlockSpec((1, 1, _BN), lambda i: (i % _NB, 0, 0))
    dspec = pl.BlockSpec((_BN, _D), lambda i: (i % _NB, 0))
    full = lambda shp: pl.BlockSpec(shp, lambda i: tuple(0 for _ in shp))

    loss = pl.pallas_call(
        _body,
        grid=(2 * _NB + _NR,),
        in_specs=[yspec, yspec, dspec, dspec],
        out_specs=pl.BlockSpec(memory_space=pltpu.SMEM),
        out_shape=jax.ShapeDtypeStruct((1, 1), jnp.float32),
        scratch_shapes=[
            pltpu.VMEM((1, _K), jnp.float32),
            pltpu.VMEM((1, _K), jnp.float32),
            pltpu.VMEM((_K, _D), jnp.float32),
            pltpu.VMEM((_K, _D), jnp.float32),
            pltpu.VMEM((_K, _K), jnp.float32),
            pltpu.VMEM((1, _K), jnp.float32),
            pltpu.VMEM((1, _K), jnp.float32),
            pltpu.SMEM((2,), jnp.float32),
        ],
        compiler_params=pltpu.CompilerParams(
            dimension_semantics=("arbitrary",)),
    )(y0, y1, data0, data1)

    return loss[0, 0]


# speedup vs baseline: 1.4923x; 1.0235x over previous
import jax
import jax.numpy as jnp
from jax.experimental import pallas as pl
from jax.experimental.pallas import tpu as pltpu

_N = 16384
_D = 64
_K = 1024
_MATCH_T = 0.1
_SIM_T = 0.5

_NB = 4
_BN = _N // _NB
_RB = 128
_NR = _K // _RB


def _body(y0_ref, y1_ref, d0_ref, d1_ref, out_ref,
          cnt0_ref, cnt1_ref, csum0_ref, csum1_ref, cooc_ref,
          rsum0_ref, rsum1_ref, acc_ref):
    i = pl.program_id(0)
    dn = (((0,), (0,)), ((), ()))

    @pl.when(i == 0)
    def _init():
        cnt0_ref[...] = jnp.zeros_like(cnt0_ref)
        cnt1_ref[...] = jnp.zeros_like(cnt1_ref)
        csum0_ref[...] = jnp.zeros_like(csum0_ref)
        csum1_ref[...] = jnp.zeros_like(csum1_ref)
        cooc_ref[...] = jnp.zeros_like(cooc_ref)
        rsum0_ref[...] = jnp.zeros_like(rsum0_ref)
        rsum1_ref[...] = jnp.zeros_like(rsum1_ref)
        acc_ref[0] = 0.0
        acc_ref[1] = 0.0

    @pl.when(i < _NB)
    def _stats():
        iota = jax.lax.broadcasted_iota(jnp.int32, (_BN, _K), 1)
        oh0 = (y0_ref[0, 0, :][:, None] == iota).astype(jnp.float32)
        oh1 = (y1_ref[0, 0, :][:, None] == iota).astype(jnp.float32)
        cnt0_ref[...] += jnp.sum(oh0, axis=0, keepdims=True)
        cnt1_ref[...] += jnp.sum(oh1, axis=0, keepdims=True)
        csum0_ref[...] += jax.lax.dot_general(
            oh0, d0_ref[...], dn, preferred_element_type=jnp.float32)
        csum1_ref[...] += jax.lax.dot_general(
            oh1, d1_ref[...], dn, preferred_element_type=jnp.float32)
        cooc_ref[...] += jax.lax.dot_general(
            oh0, oh1, dn, preferred_element_type=jnp.float32)

    @pl.when(jnp.logical_and(i >= _NB, i < 2 * _NB))
    def _radius():
        iota = jax.lax.broadcasted_iota(jnp.int32, (_BN, _K), 1)

        def one_view(y_ref, d_ref, cnt_ref, csum_ref, rsum_ref):
            rc = 1.0 / jnp.clip(cnt_ref[...], 1e-6, None)
            centers = csum_ref[...] * rc.T
            oh = (y_ref[0, 0, :][:, None] == iota).astype(jnp.float32)
            gathered = jnp.dot(oh, centers,
                               preferred_element_type=jnp.float32)
            diff = d_ref[...] - gathered
            d = jnp.sqrt(jnp.sum(diff * diff, axis=1, keepdims=True))
            rsum_ref[...] += jax.lax.dot_general(
                d, oh, dn, preferred_element_type=jnp.float32
            ).reshape(1, _K)

        one_view(y0_ref, d0_ref, cnt0_ref, csum0_ref, rsum0_ref)
        one_view(y1_ref, d1_ref, cnt1_ref, csum1_ref, rsum1_ref)

    @pl.when(i >= 2 * _NB)
    def _loss():
        pi = i - 2 * _NB
        cnt0 = cnt0_ref[...]
        cnt1 = cnt1_ref[...]
        rc0 = 1.0 / jnp.clip(cnt0, 1e-6, None)
        rc1 = 1.0 / jnp.clip(cnt1, 1e-6, None)
        centers0 = csum0_ref[...] * rc0.T
        centers1 = csum1_ref[...] * rc1.T
        rs0 = rsum0_ref[...] * rc0
        rs1 = rsum1_ref[...] * rc1

        x = jnp.concatenate([centers0, centers1], axis=0)
        nx = jnp.sqrt(jnp.sum(x * x, axis=1, keepdims=True))

        a0 = pi * _RB

        def half(centers, rs, csum_ref, rsum_ref, cnt_blk_ref, half_idx):
            cntb = cnt_blk_ref[:, pl.ds(a0, _RB)]
            rcb = 1.0 / jnp.clip(cntb, 1e-6, None)
            cb = csum_ref[pl.ds(a0, _RB), :] * rcb.T
            rsb = rsum_ref[:, pl.ds(a0, _RB)] * rcb

            aa = jnp.sum(cb * cb, axis=1, keepdims=True)
            bb = jnp.sum(centers * centers, axis=1,
                         keepdims=True).T
            sq = aa + bb - 2.0 * jnp.dot(
                cb, centers.T, preferred_element_type=jnp.float32)
            dist = jnp.sqrt(jnp.clip(sq, 0.0, None))
            extra = rsb.T + rs
            geo = dist <= extra
            cnorm = jnp.clip(jnp.sqrt(bb), 1e-8, None)
            cnb = cb / jnp.clip(jnp.sqrt(aa), 1e-8, None)
            sim = jnp.dot(cnb, (centers / cnorm.T).T,
                          preferred_element_type=jnp.float32)
            mask = jnp.logical_and(geo, sim > _SIM_T).astype(jnp.float32)

            ca = cntb.T
            if half_idx == 0:
                coocb = cooc_ref[pl.ds(a0, _RB), :]
                m = jnp.minimum(ca, cnt1)
            else:
                coocb = cooc_ref[:, pl.ds(a0, _RB)].T
                m = jnp.minimum(ca, cnt0)
            m = jnp.where(m == 0.0, 1.0, m)
            inter = (coocb / m > _MATCH_T).astype(jnp.float32)

            if half_idx == 0:
                posb = jnp.concatenate([mask, inter], axis=1)
            else:
                posb = jnp.concatenate([inter, mask], axis=1)

            grow = a0 + half_idx * _K + jax.lax.broadcasted_iota(
                jnp.int32, (_RB, 2 * _K), 0)
            gcol = jax.lax.broadcasted_iota(jnp.int32, (_RB, 2 * _K), 1)
            posb = jnp.where(gcol == grow, 0.0, posb)

            nxb = jnp.sqrt(aa)
            num = jnp.dot(cb, x.T, preferred_element_type=jnp.float32)
            logits = num / (nxb * nx.T + 1e-12)
            mx = jnp.max(logits, axis=1, keepdims=True)
            lse = mx + jnp.log(jnp.sum(jnp.exp(logits - mx), axis=1,
                                       keepdims=True))
            log_prob = logits - lse

            pc = jnp.sum(posb, axis=1)
            valid = pc > 0.0
            rl = -jnp.sum(log_prob * posb, axis=1) / jnp.clip(pc, 1.0,
                                                              None)
            contrib = jnp.sum(jnp.where(valid, rl, 0.0))
            nv = jnp.sum(valid.astype(jnp.float32))
            return contrib, nv

        c0, n0 = half(centers0, rs0, csum0_ref, rsum0_ref, cnt0_ref, 0)
        c1, n1 = half(centers1, rs1, csum1_ref, rsum1_ref, cnt1_ref, 1)
        acc_ref[0] += c0 + c1
        acc_ref[1] += n0 + n1

    @pl.when(i == 2 * _NB + _NR - 1)
    def _fin():
        out_ref[0, 0] = acc_ref[0] / jnp.maximum(acc_ref[1], 1.0)


def kernel(data0, data1, y_parts0, y_parts1):
    y0 = y_parts0.reshape(_NB, 1, _BN)
    y1 = y_parts1.reshape(_NB, 1, _BN)

    yspec = pl.BlockSpec((1, 1, _BN), lambda i: (i % _NB, 0, 0))
    dspec = pl.BlockSpec((_BN, _D), lambda i: (i % _NB, 0))
    full = lambda shp: pl.BlockSpec(shp, lambda i: tuple(0 for _ in shp))

    loss = pl.pallas_call(
        _body,
        grid=(2 * _NB + _NR,),
        in_specs=[yspec, yspec, dspec, dspec],
        out_specs=pl.BlockSpec(memory_space=pltpu.SMEM),
        out_shape=jax.ShapeDtypeStruct((1, 1), jnp.float32),
        scratch_shapes=[
            pltpu.VMEM((1, _K), jnp.float32),
            pltpu.VMEM((1, _K), jnp.float32),
            pltpu.VMEM((_K, _D), jnp.float32),
            pltpu.VMEM((_K, _D), jnp.float32),
            pltpu.VMEM((_K, _K), jnp.float32),
            pltpu.VMEM((1, _K), jnp.float32),
            pltpu.VMEM((1, _K), jnp.float32),
            pltpu.SMEM((2,), jnp.float32),
        ],
        compiler_params=pltpu.CompilerParams(
            dimension_semantics=("arbitrary",)),
    )(y0, y1, data0, data1)

    return loss[0, 0]
